# Initial kernel scaffold; baseline (speedup 1.0000x reference)
import jax
import jax.numpy as jnp
from jax.experimental import pallas as pl


def kernel(inputs, nv_in):
    raise NotImplementedError("write your pallas kernel here")


# speedup vs baseline: 2.9204x; 2.9204x over previous
import functools

import jax
import jax.numpy as jnp
from jax import lax
from jax.experimental import pallas as pl
from jax.experimental.pallas import tpu as pltpu
from jax.experimental.pallas import tpu_sc as plsc

B = 16
D = 128
TOTAL = 32768
NW = 32
RPW = TOTAL // NW
CH = 128
NCH = RPW // CH
LANES = 16
G = D // LANES


def _sc_partial_sums(x_flat, starts):
    mesh = plsc.VectorSubcoreMesh(core_axis_name="c", subcore_axis_name="s")

    @functools.partial(
        pl.kernel,
        mesh=mesh,
        out_type=jax.ShapeDtypeStruct((NW * B * D,), jnp.float32),
        scratch_types=[
            pltpu.VMEM((B,), jnp.int32),
            pltpu.VMEM((CH * D,), jnp.float32),
            pltpu.VMEM((B * D,), jnp.float32),
        ],
        compiler_params=pltpu.CompilerParams(needs_layout_passes=False),
    )
    def k(x_hbm, starts_hbm, part_hbm, starts_v, buf_v, acc_v):
        wid = lax.axis_index("s") * 2 + lax.axis_index("c")

        pltpu.sync_copy(starts_hbm, starts_v)
        starts_vec = starts_v[...]
        lanes = lax.iota(jnp.int32, LANES)
        goff = [g * LANES + lanes for g in range(G)]

        zero = jnp.zeros((LANES,), jnp.float32)
        for i in range(B * D // LANES):
            acc_v[pl.ds(i * LANES, LANES)] = zero

        row_lo = wid * RPW

        def chunk_body(kk, carry):
            base = row_lo + kk * CH
            pltpu.sync_copy(x_hbm.at[pl.ds(base * D, CH * D)], buf_v)

            def row_body(rr, c):
                r = base + rr
                seg = plsc.all_reduce_population_count(starts_vec <= r) - 1
                idx_base = seg * D
                for g in range(G):
                    vals = buf_v[pl.ds(rr * D + g * LANES, LANES)]
                    plsc.addupdate_scatter(acc_v, [idx_base + goff[g]], vals)
                return c

            lax.fori_loop(0, CH, row_body, 0)
            return carry

        lax.fori_loop(0, NCH, chunk_body, 0)
        pltpu.sync_copy(acc_v, part_hbm.at[pl.ds(wid * B * D, B * D)])

    return k(x_flat, starts)


def _tc_combine(partials, counts):
    def body(p_ref, c_ref, o_ref):
        acc = p_ref[0]
        for i in range(1, NW):
            acc = acc + p_ref[i]
        o_ref[...] = acc / c_ref[...]

    return pl.pallas_call(
        body,
        out_shape=jax.ShapeDtypeStruct((B, D), jnp.float32),
    )(partials, counts)


def kernel(inputs, nv_in):
    x_flat = inputs.reshape(-1)
    csum = jnp.cumsum(nv_in)
    starts = (csum - nv_in).astype(jnp.int32)
    part = _sc_partial_sums(x_flat, starts)
    partials = part.reshape(NW, B, D)
    counts = jnp.maximum(nv_in.astype(jnp.float32), 1.0).reshape(B, 1)
    return _tc_combine(partials, counts)


# speedup vs baseline: 4.9395x; 1.6913x over previous
import functools

import jax
import jax.numpy as jnp
from jax import lax
from jax.experimental import pallas as pl
from jax.experimental.pallas import tpu as pltpu
from jax.experimental.pallas import tpu_sc as plsc

B = 16
D = 128
TOTAL = 32768
NW = 32
RPW = TOTAL // NW
CH = 128
NCH = RPW // CH
LANES = 16
G = D // LANES


def _sc_partial_sums(x_flat, starts):
    mesh = plsc.VectorSubcoreMesh(core_axis_name="c", subcore_axis_name="s")

    @functools.partial(
        pl.kernel,
        mesh=mesh,
        out_type=jax.ShapeDtypeStruct((NW * B * D,), jnp.float32),
        scratch_types=[
            pltpu.VMEM((B,), jnp.int32),
            pltpu.VMEM((CH * D,), jnp.float32),
            pltpu.VMEM((CH * D,), jnp.float32),
            pltpu.VMEM((B * D,), jnp.float32),
            pltpu.SemaphoreType.DMA,
            pltpu.SemaphoreType.DMA,
        ],
        compiler_params=pltpu.CompilerParams(needs_layout_passes=False),
    )
    def k(x_hbm, starts_hbm, part_hbm, starts_v, buf0_v, buf1_v, acc_v,
          sem0, sem1):
        wid = lax.axis_index("s") * 2 + lax.axis_index("c")

        pltpu.sync_copy(starts_hbm, starts_v)
        starts_vec = starts_v[...]
        lanes = lax.iota(jnp.int32, LANES)
        goff = [g * LANES + lanes for g in range(G)]

        zero = jnp.zeros((LANES,), jnp.float32)
        for i in range(B * D // LANES):
            acc_v[pl.ds(i * LANES, LANES)] = zero

        row_lo = wid * RPW
        bufs = [buf0_v, buf1_v]
        sems = [sem0, sem1]

        def start_chunk(kk):
            base = row_lo + kk * CH
            return pltpu.async_copy(
                x_hbm.at[pl.ds(base * D, CH * D)], bufs[kk % 2], sems[kk % 2])

        pending = start_chunk(0)
        for kk in range(NCH):
            base = row_lo + kk * CH
            buf_v = bufs[kk % 2]
            pending.wait()
            if kk + 1 < NCH:
                pending = start_chunk(kk + 1)

            @plsc.parallel_loop(0, CH, unroll=2)
            def row_body(rr):
                r = base + rr
                seg = plsc.all_reduce_population_count(starts_vec <= r) - 1
                idx_base = seg * D
                for g in range(G):
                    vals = buf_v[pl.ds(rr * D + g * LANES, LANES)]
                    plsc.addupdate_scatter(acc_v, [idx_base + goff[g]], vals)

        pltpu.sync_copy(acc_v, part_hbm.at[pl.ds(wid * B * D, B * D)])

    return k(x_flat, starts)


def _tc_combine(partials, counts):
    def body(p_ref, c_ref, o_ref):
        acc = p_ref[0]
        for i in range(1, NW):
            acc = acc + p_ref[i]
        o_ref[...] = acc / c_ref[...]

    return pl.pallas_call(
        body,
        out_shape=jax.ShapeDtypeStruct((B, D), jnp.float32),
    )(partials, counts)


def kernel(inputs, nv_in):
    x_flat = inputs.reshape(-1)
    csum = jnp.cumsum(nv_in)
    starts = (csum - nv_in).astype(jnp.int32)
    part = _sc_partial_sums(x_flat, starts)
    partials = part.reshape(NW, B, D)
    counts = jnp.maximum(nv_in.astype(jnp.float32), 1.0).reshape(B, 1)
    return _tc_combine(partials, counts)
